# baseline (speedup 1.0000x reference)
import functools

import jax
import jax.numpy as jnp
from jax import lax
from jax.experimental import pallas as pl
from jax.experimental.pallas import tpu as pltpu
from jax.experimental.pallas import tpu_sc as plsc

_NUM_CORES = 2
_NUM_SUBCORES = 16
_NUM_WORKERS = _NUM_CORES * _NUM_SUBCORES
_CHUNK = 128
_HBM_CHUNKS = 2


@functools.lru_cache(maxsize=None)
def _make_lookup(vocab, dim, batch):
    assert batch % (_NUM_WORKERS * _CHUNK) == 0
    nchunks = batch // (_NUM_WORKERS * _CHUNK)
    mesh = plsc.VectorSubcoreMesh(core_axis_name="c", subcore_axis_name="s")

    @functools.partial(
        pl.kernel,
        mesh=mesh,
        out_type=jax.ShapeDtypeStruct((_NUM_WORKERS, nchunks, _CHUNK, dim),
                                      jnp.float32),
        scratch_types=[
            pltpu.VMEM((nchunks, _CHUNK), jnp.int32),
            pltpu.VMEM((nchunks, _CHUNK, dim), jnp.float32),
            pltpu.VMEM_SHARED((vocab, dim), jnp.float32),
        ] + [pltpu.SemaphoreType.DMA] * (nchunks + 1),
    )
    def lookup(table_hbm, idx_hbm, out_hbm, idx_v, rows_v, table_sp, *sems):
        gsems, wsem = sems[:nchunks], sems[nchunks]
        sid = lax.axis_index("s")
        wid = sid * _NUM_CORES + lax.axis_index("c")
        pltpu.sync_copy(idx_hbm.at[wid], idx_v)
        gathers = [
            pltpu.async_copy(table_hbm.at[idx_v.at[j]], rows_v.at[j], gsems[j])
            for j in range(_HBM_CHUNKS)
        ]
        @pl.when(sid == 0)
        def _stage_table():
            pltpu.sync_copy(table_hbm, table_sp)
        plsc.subcore_barrier()
        gathers += [
            pltpu.async_copy(table_sp.at[idx_v.at[j]], rows_v.at[j], gsems[j])
            for j in range(_HBM_CHUNKS, nchunks)
        ]
        writes = []
        for j in range(nchunks):
            gathers[j].wait()
            writes.append(
                pltpu.async_copy(rows_v.at[j], out_hbm.at[wid, j], wsem))
        for w in writes:
            w.wait()

    return lookup


def kernel(level, table):
    (batch,) = level.shape
    vocab, dim = table.shape
    nchunks = batch // (_NUM_WORKERS * _CHUNK)
    idx = level.astype(jnp.int32).reshape(_NUM_WORKERS, nchunks, _CHUNK)
    out = _make_lookup(vocab, dim, batch)(table, idx)
    return out.reshape(batch, dim)


# speedup vs baseline: 1.0445x; 1.0445x over previous
import functools

import jax
import jax.numpy as jnp
from jax import lax
from jax.experimental import pallas as pl
from jax.experimental.pallas import tpu as pltpu
from jax.experimental.pallas import tpu_sc as plsc

_NUM_CORES = 2
_NUM_SUBCORES = 16
_NUM_WORKERS = _NUM_CORES * _NUM_SUBCORES
_CHUNK = 128


@functools.lru_cache(maxsize=None)
def _make_lookup(vocab, dim, batch):
    assert batch % (_NUM_WORKERS * _CHUNK) == 0
    nchunks = batch // (_NUM_WORKERS * _CHUNK)
    mesh = plsc.VectorSubcoreMesh(core_axis_name="c", subcore_axis_name="s")

    @functools.partial(
        pl.kernel,
        mesh=mesh,
        out_type=jax.ShapeDtypeStruct((_NUM_WORKERS, nchunks, _CHUNK, dim),
                                      jnp.float32),
        scratch_types=[
            pltpu.VMEM((nchunks, _CHUNK), jnp.int32),
            pltpu.VMEM((nchunks, _CHUNK, dim), jnp.float32),
            pltpu.VMEM_SHARED((vocab, dim), jnp.float32),
        ] + [pltpu.SemaphoreType.DMA] * (nchunks + 1),
    )
    def lookup(table_hbm, idx_hbm, out_hbm, idx_v, rows_v, table_sp, *sems):
        gsems, wsem = sems[:nchunks], sems[nchunks]
        sid = lax.axis_index("s")
        wid = sid * _NUM_CORES + lax.axis_index("c")
        for k in range((vocab + 127) // 128):
            off = k * 128
            sz = min(128, vocab - off)
            @pl.when(sid == k)
            def _stage_table(off=off, sz=sz):
                pltpu.sync_copy(table_hbm.at[pl.ds(off, sz)],
                                table_sp.at[pl.ds(off, sz)])
        pltpu.sync_copy(idx_hbm.at[wid], idx_v)
        plsc.subcore_barrier()
        gathers = [
            pltpu.async_copy(table_sp.at[idx_v.at[j]], rows_v.at[j], gsems[j])
            for j in range(nchunks)
        ]
        writes = []
        for j in range(nchunks):
            gathers[j].wait()
            writes.append(
                pltpu.async_copy(rows_v.at[j], out_hbm.at[wid, j], wsem))
        for w in writes:
            w.wait()

    return lookup


def kernel(level, table):
    (batch,) = level.shape
    vocab, dim = table.shape
    nchunks = batch // (_NUM_WORKERS * _CHUNK)
    idx = level.astype(jnp.int32).reshape(_NUM_WORKERS, nchunks, _CHUNK)
    out = _make_lookup(vocab, dim, batch)(table, idx)
    return out.reshape(batch, dim)


# speedup vs baseline: 1.0590x; 1.0138x over previous
import functools

import jax
import jax.numpy as jnp
from jax import lax
from jax.experimental import pallas as pl
from jax.experimental.pallas import tpu as pltpu
from jax.experimental.pallas import tpu_sc as plsc

_NUM_CORES = 2
_NUM_SUBCORES = 16
_NUM_WORKERS = _NUM_CORES * _NUM_SUBCORES
_CHUNK = 64


@functools.lru_cache(maxsize=None)
def _make_lookup(vocab, dim, batch):
    assert batch % (_NUM_WORKERS * _CHUNK) == 0
    nchunks = batch // (_NUM_WORKERS * _CHUNK)
    mesh = plsc.VectorSubcoreMesh(core_axis_name="c", subcore_axis_name="s")

    @functools.partial(
        pl.kernel,
        mesh=mesh,
        out_type=jax.ShapeDtypeStruct((_NUM_WORKERS, nchunks, _CHUNK, dim),
                                      jnp.float32),
        scratch_types=[
            pltpu.VMEM((nchunks, _CHUNK), jnp.int32),
            pltpu.VMEM((nchunks, _CHUNK, dim), jnp.float32),
            pltpu.VMEM_SHARED((vocab, dim), jnp.float32),
        ] + [pltpu.SemaphoreType.DMA] * (nchunks + 1),
    )
    def lookup(table_hbm, idx_hbm, out_hbm, idx_v, rows_v, table_sp, *sems):
        gsems, wsem = sems[:nchunks], sems[nchunks]
        sid = lax.axis_index("s")
        wid = sid * _NUM_CORES + lax.axis_index("c")
        for k in range((vocab + 127) // 128):
            off = k * 128
            sz = min(128, vocab - off)
            @pl.when(sid == k)
            def _stage_table(off=off, sz=sz):
                pltpu.sync_copy(table_hbm.at[pl.ds(off, sz)],
                                table_sp.at[pl.ds(off, sz)])
        pltpu.sync_copy(idx_hbm.at[wid], idx_v)
        plsc.subcore_barrier()
        gathers = [
            pltpu.async_copy(table_sp.at[idx_v.at[j]], rows_v.at[j], gsems[j])
            for j in range(nchunks)
        ]
        writes = []
        for j in range(nchunks):
            gathers[j].wait()
            writes.append(
                pltpu.async_copy(rows_v.at[j], out_hbm.at[wid, j], wsem))
        for w in writes:
            w.wait()

    return lookup


def kernel(level, table):
    (batch,) = level.shape
    vocab, dim = table.shape
    nchunks = batch // (_NUM_WORKERS * _CHUNK)
    idx = level.astype(jnp.int32).reshape(_NUM_WORKERS, nchunks, _CHUNK)
    out = _make_lookup(vocab, dim, batch)(table, idx)
    return out.reshape(batch, dim)
